# baseline (speedup 1.0000x reference)
import functools

import jax
import jax.numpy as jnp
from jax import lax
from jax.experimental import pallas as pl
from jax.experimental.pallas import tpu as pltpu
from jax.experimental.pallas import tpu_sc as plsc

N = 10000
E = 320000
D = 128
G = 64

NC = 2
NS = 16
NW = NC * NS
CHUNK = 128
CPW = 80
HALF = 40
NBUF = 2
EPW = CPW * CHUNK
E_PAD = NW * EPW
ACC_ROWS = 10240
DEAD_ROW = N + 8
ZROWS = 128


def _sc_aggregate_build():
    mesh = plsc.VectorSubcoreMesh(core_axis_name="c", subcore_axis_name="s")

    @functools.partial(
        pl.kernel,
        out_type=jax.ShapeDtypeStruct((NC, ACC_ROWS, D), jnp.float32),
        mesh=mesh,
        scratch_types=[
            pltpu.VMEM((2, CHUNK), jnp.int32),
            pltpu.VMEM((2, CHUNK), jnp.int32),
            pltpu.VMEM((2, CHUNK, D), jnp.float32),
            pltpu.VMEM_SHARED((ACC_ROWS, D), jnp.float32),
            pltpu.SemaphoreType.DMA,
            pltpu.SemaphoreType.DMA,
            pltpu.SemaphoreType.DMA,
            pltpu.SemaphoreType.DMA,
            pltpu.SemaphoreType.DMA,
            pltpu.SemaphoreType.DMA,
        ],
    )
    def sc_aggregate(src_hbm, dst_hbm, h_hbm, out_hbm,
                     srcb, dstb, rows, acc, g0, g1, ss0, ss1, ds0, ds1):
        cid = lax.axis_index("c")
        sid = lax.axis_index("s")
        wid = sid * NC + cid
        gsem = (g0, g1)
        ssem = (ss0, ss1)
        dsem = (ds0, ds1)

        zero16 = jnp.zeros((16,), jnp.float32)

        @pl.loop(0, ZROWS)
        def _(r):
            for j in range(D // 16):
                rows[0, r, pl.ds(j * 16, 16)] = zero16

        rows_per_tile = ACC_ROWS // NS

        @pl.loop(0, rows_per_tile // ZROWS)
        def _(z):
            pltpu.sync_copy(
                rows.at[0],
                acc.at[pl.ds(sid * rows_per_tile + z * ZROWS, ZROWS)])

        plsc.subcore_barrier()

        ebase = wid * EPW

        def fetch_idx(c, b, sync):
            off = pl.ds(ebase + c * CHUNK, CHUNK)
            if sync:
                pltpu.sync_copy(src_hbm.at[off], srcb.at[b])
                pltpu.sync_copy(dst_hbm.at[off], dstb.at[b])
            else:
                pltpu.async_copy(src_hbm.at[off], srcb.at[b], ssem[b])
                pltpu.async_copy(dst_hbm.at[off], dstb.at[b], dsem[b])

        def wait_idx(b):
            off = pl.ds(0, CHUNK)
            pltpu.make_async_copy(src_hbm.at[off], srcb.at[b], ssem[b]).wait()
            pltpu.make_async_copy(dst_hbm.at[off], dstb.at[b], dsem[b]).wait()

        def issue_gather(b):
            pltpu.async_copy(h_hbm.at[srcb.at[b]], rows.at[b], gsem[b])

        def wait_gather(b):
            pltpu.make_async_copy(
                h_hbm.at[srcb.at[b]], rows.at[b], gsem[b]).wait()

        def scatter(b):
            pltpu.sync_copy(rows.at[b], acc.at[dstb.at[b]], add=True)

        @pl.loop(0, CPW)
        def _(c):
            fetch_idx(c, 0, True)
            issue_gather(0)
            wait_gather(0)
            scatter(0)

        plsc.subcore_barrier()

        pltpu.sync_copy(
            acc.at[pl.ds(sid * rows_per_tile, rows_per_tile)],
            out_hbm.at[cid, pl.ds(sid * rows_per_tile, rows_per_tile)])

    return sc_aggregate


_sc_aggregate = _sc_aggregate_build()


def _layer_body(relu, p_ref, h_ref, wrel_ref, b_ref, wroot_ref, o_ref):
    aggr = p_ref[0] + p_ref[1]
    out = jnp.dot(aggr, wrel_ref[...], preferred_element_type=jnp.float32)
    out = out + jnp.dot(h_ref[...], wroot_ref[...],
                        preferred_element_type=jnp.float32)
    out = out + b_ref[...]
    if relu:
        out = jnp.maximum(out, 0.0)
    o_ref[...] = out


def _tc_layer(P, h, W_rel, b, W_root, relu):
    blk = 1000
    nblk = N // blk
    return pl.pallas_call(
        functools.partial(_layer_body, relu),
        grid=(nblk,),
        in_specs=[
            pl.BlockSpec((NC, blk, D), lambda i: (0, i, 0)),
            pl.BlockSpec((blk, D), lambda i: (i, 0)),
            pl.BlockSpec((D, D), lambda i: (0, 0)),
            pl.BlockSpec((1, D), lambda i: (0, 0)),
            pl.BlockSpec((D, D), lambda i: (0, 0)),
        ],
        out_specs=pl.BlockSpec((blk, D), lambda i: (i, 0)),
        out_shape=jax.ShapeDtypeStruct((N, D), jnp.float32),
    )(P, h, W_rel, b.reshape(1, D), W_root)


def _pool_body(nblk, batch_ref, h_ref, wl_ref, bl_ref, o_ref, acc_ref, cnt_ref):
    i = pl.program_id(0)

    @pl.when(i == 0)
    def _():
        acc_ref[...] = jnp.zeros_like(acc_ref)
        cnt_ref[...] = jnp.zeros_like(cnt_ref)

    ids = batch_ref[0, 0, :]
    blk = ids.shape[0]
    onehot = (ids[:, None] ==
              lax.broadcasted_iota(jnp.int32, (blk, G), 1)).astype(jnp.float32)
    acc_ref[...] += lax.dot_general(
        onehot, h_ref[...], (((0,), (0,)), ((), ())),
        preferred_element_type=jnp.float32)
    cnt_ref[...] += jnp.sum(onehot, axis=0)[:, None]

    @pl.when(i == nblk - 1)
    def _():
        pooled = acc_ref[...] / jnp.maximum(cnt_ref[...], 1.0)
        o_ref[...] = jnp.dot(pooled, wl_ref[...],
                             preferred_element_type=jnp.float32) + bl_ref[...]


def _tc_pool(h, batch, Wl, bl):
    blk = 1000
    nblk = N // blk
    C = Wl.shape[1]
    return pl.pallas_call(
        functools.partial(_pool_body, nblk),
        grid=(nblk,),
        in_specs=[
            pl.BlockSpec((1, 1, blk), lambda i: (i, 0, 0)),
            pl.BlockSpec((blk, D), lambda i: (i, 0)),
            pl.BlockSpec((D, C), lambda i: (0, 0)),
            pl.BlockSpec((1, C), lambda i: (0, 0)),
        ],
        out_specs=pl.BlockSpec((G, C), lambda i: (0, 0)),
        out_shape=jax.ShapeDtypeStruct((G, C), jnp.float32),
        scratch_shapes=[
            pltpu.VMEM((G, D), jnp.float32),
            pltpu.VMEM((G, D), jnp.float32),
        ],
    )(batch.reshape(N // blk, 1, blk), h, Wl, bl.reshape(1, C))


def kernel(x, edge_index, batch, W1_rel, b1, W1_root, W2_rel, b2, W2_root,
           W3_rel, b3, W3_root, Wl, bl):
    pad = E_PAD - E
    src = jnp.concatenate([edge_index[0], jnp.zeros((pad,), jnp.int32)])
    dst = jnp.concatenate(
        [edge_index[1], jnp.full((pad,), DEAD_ROW, jnp.int32)])

    h = x
    for W_rel, b, W_root, relu in (
            (W1_rel, b1, W1_root, True),
            (W2_rel, b2, W2_root, True),
            (W3_rel, b3, W3_root, False)):
        P = _sc_aggregate(src, dst, h)
        h = _tc_layer(P, h, W_rel, b, W_root, relu)

    return _tc_pool(h, batch, Wl, bl)


# speedup vs baseline: 1.0003x; 1.0003x over previous
import functools

import jax
import jax.numpy as jnp
from jax import lax
from jax.experimental import pallas as pl
from jax.experimental.pallas import tpu as pltpu
from jax.experimental.pallas import tpu_sc as plsc

N = 10000
E = 320000
D = 128
G = 64

NC = 2
NS = 16
NW = NC * NS
CHUNK = 128
CPW = 80
HALF = 40
NBUF = 2
EPW = CPW * CHUNK
E_PAD = NW * EPW
ACC_ROWS = 10240
DEAD_ROW = N + 8
ZROWS = 128


def _sc_aggregate_build():
    mesh = plsc.VectorSubcoreMesh(core_axis_name="c", subcore_axis_name="s")

    @functools.partial(
        pl.kernel,
        out_type=jax.ShapeDtypeStruct((NC, ACC_ROWS, D), jnp.float32),
        mesh=mesh,
        scratch_types=[
            pltpu.VMEM((2, CHUNK), jnp.int32),
            pltpu.VMEM((2, CHUNK), jnp.int32),
            pltpu.VMEM((2, CHUNK, D), jnp.float32),
            pltpu.VMEM_SHARED((ACC_ROWS, D), jnp.float32),
            pltpu.SemaphoreType.DMA,
            pltpu.SemaphoreType.DMA,
            pltpu.SemaphoreType.DMA,
            pltpu.SemaphoreType.DMA,
            pltpu.SemaphoreType.DMA,
            pltpu.SemaphoreType.DMA,
        ],
    )
    def sc_aggregate(src_hbm, dst_hbm, h_hbm, out_hbm,
                     srcb, dstb, rows, acc, g0, g1, ss0, ss1, ds0, ds1):
        cid = lax.axis_index("c")
        sid = lax.axis_index("s")
        wid = sid * NC + cid
        gsem = (g0, g1)
        ssem = (ss0, ss1)
        dsem = (ds0, ds1)

        zero16 = jnp.zeros((16,), jnp.float32)

        @pl.loop(0, ZROWS)
        def _(r):
            for j in range(D // 16):
                rows[0, r, pl.ds(j * 16, 16)] = zero16

        rows_per_tile = ACC_ROWS // NS

        @pl.loop(0, rows_per_tile // ZROWS)
        def _(z):
            pltpu.sync_copy(
                rows.at[0],
                acc.at[pl.ds(sid * rows_per_tile + z * ZROWS, ZROWS)])

        plsc.subcore_barrier()

        ebase = wid * EPW

        def fetch_idx(c, b, sync):
            off = pl.ds(ebase + c * CHUNK, CHUNK)
            if sync:
                pltpu.sync_copy(src_hbm.at[off], srcb.at[b])
                pltpu.sync_copy(dst_hbm.at[off], dstb.at[b])
            else:
                pltpu.async_copy(src_hbm.at[off], srcb.at[b], ssem[b])
                pltpu.async_copy(dst_hbm.at[off], dstb.at[b], dsem[b])

        def wait_idx(b):
            off = pl.ds(0, CHUNK)
            pltpu.make_async_copy(src_hbm.at[off], srcb.at[b], ssem[b]).wait()
            pltpu.make_async_copy(dst_hbm.at[off], dstb.at[b], dsem[b]).wait()

        def issue_gather(b):
            pltpu.async_copy(h_hbm.at[srcb.at[b]], rows.at[b], gsem[b])

        def wait_gather(b):
            pltpu.make_async_copy(
                h_hbm.at[srcb.at[b]], rows.at[b], gsem[b]).wait()

        def scatter(b):
            pltpu.sync_copy(rows.at[b], acc.at[dstb.at[b]], add=True)

        @pl.loop(0, CPW)
        def _(c):
            base = ebase + c * CHUNK
            pltpu.sync_copy(src_hbm.at[pl.ds(base, CHUNK)], srcb.at[0])
            pltpu.sync_copy(dst_hbm.at[pl.ds(base, CHUNK)], dstb.at[0])
            pltpu.async_copy(h_hbm.at[srcb.at[0]], rows.at[0], gsem[0]).wait()
            pltpu.sync_copy(rows.at[0], acc.at[dstb.at[0]], add=True)

        plsc.subcore_barrier()

        pltpu.sync_copy(
            acc.at[pl.ds(sid * rows_per_tile, rows_per_tile)],
            out_hbm.at[cid, pl.ds(sid * rows_per_tile, rows_per_tile)])

    return sc_aggregate


_sc_aggregate = _sc_aggregate_build()


def _layer_body(relu, p_ref, h_ref, wrel_ref, b_ref, wroot_ref, o_ref):
    aggr = p_ref[0] + p_ref[1]
    out = jnp.dot(aggr, wrel_ref[...], preferred_element_type=jnp.float32)
    out = out + jnp.dot(h_ref[...], wroot_ref[...],
                        preferred_element_type=jnp.float32)
    out = out + b_ref[...]
    if relu:
        out = jnp.maximum(out, 0.0)
    o_ref[...] = out


def _tc_layer(P, h, W_rel, b, W_root, relu):
    blk = 1000
    nblk = N // blk
    return pl.pallas_call(
        functools.partial(_layer_body, relu),
        grid=(nblk,),
        in_specs=[
            pl.BlockSpec((NC, blk, D), lambda i: (0, i, 0)),
            pl.BlockSpec((blk, D), lambda i: (i, 0)),
            pl.BlockSpec((D, D), lambda i: (0, 0)),
            pl.BlockSpec((1, D), lambda i: (0, 0)),
            pl.BlockSpec((D, D), lambda i: (0, 0)),
        ],
        out_specs=pl.BlockSpec((blk, D), lambda i: (i, 0)),
        out_shape=jax.ShapeDtypeStruct((N, D), jnp.float32),
    )(P, h, W_rel, b.reshape(1, D), W_root)


def _pool_body(nblk, batch_ref, h_ref, wl_ref, bl_ref, o_ref, acc_ref, cnt_ref):
    i = pl.program_id(0)

    @pl.when(i == 0)
    def _():
        acc_ref[...] = jnp.zeros_like(acc_ref)
        cnt_ref[...] = jnp.zeros_like(cnt_ref)

    ids = batch_ref[0, 0, :]
    blk = ids.shape[0]
    onehot = (ids[:, None] ==
              lax.broadcasted_iota(jnp.int32, (blk, G), 1)).astype(jnp.float32)
    acc_ref[...] += lax.dot_general(
        onehot, h_ref[...], (((0,), (0,)), ((), ())),
        preferred_element_type=jnp.float32)
    cnt_ref[...] += jnp.sum(onehot, axis=0)[:, None]

    @pl.when(i == nblk - 1)
    def _():
        pooled = acc_ref[...] / jnp.maximum(cnt_ref[...], 1.0)
        o_ref[...] = jnp.dot(pooled, wl_ref[...],
                             preferred_element_type=jnp.float32) + bl_ref[...]


def _tc_pool(h, batch, Wl, bl):
    blk = 1000
    nblk = N // blk
    C = Wl.shape[1]
    return pl.pallas_call(
        functools.partial(_pool_body, nblk),
        grid=(nblk,),
        in_specs=[
            pl.BlockSpec((1, 1, blk), lambda i: (i, 0, 0)),
            pl.BlockSpec((blk, D), lambda i: (i, 0)),
            pl.BlockSpec((D, C), lambda i: (0, 0)),
            pl.BlockSpec((1, C), lambda i: (0, 0)),
        ],
        out_specs=pl.BlockSpec((G, C), lambda i: (0, 0)),
        out_shape=jax.ShapeDtypeStruct((G, C), jnp.float32),
        scratch_shapes=[
            pltpu.VMEM((G, D), jnp.float32),
            pltpu.VMEM((G, D), jnp.float32),
        ],
    )(batch.reshape(N // blk, 1, blk), h, Wl, bl.reshape(1, C))


def kernel(x, edge_index, batch, W1_rel, b1, W1_root, W2_rel, b2, W2_root,
           W3_rel, b3, W3_root, Wl, bl):
    pad = E_PAD - E
    src = jnp.concatenate([edge_index[0], jnp.zeros((pad,), jnp.int32)])
    dst = jnp.concatenate(
        [edge_index[1], jnp.full((pad,), DEAD_ROW, jnp.int32)])

    h = x
    for W_rel, b, W_root, relu in (
            (W1_rel, b1, W1_root, True),
            (W2_rel, b2, W2_root, True),
            (W3_rel, b3, W3_root, False)):
        P = _sc_aggregate(src, dst, h)
        h = _tc_layer(P, h, W_rel, b, W_root, relu)

    return _tc_pool(h, batch, Wl, bl)


# speedup vs baseline: 1.4506x; 1.4502x over previous
import functools

import jax
import jax.numpy as jnp
from jax import lax
from jax.experimental import pallas as pl
from jax.experimental.pallas import tpu as pltpu
from jax.experimental.pallas import tpu_sc as plsc

N = 10000
E = 320000
D = 128
G = 64

NC = 2
NS = 16
NW = NC * NS
CHUNK = 128
CPW = 79
EPW = CPW * CHUNK
E_PAD = NW * EPW
ACC_ROWS = 10240
DEAD_ROW = N + 8
ZROWS = 128


def _sc_aggregate_build():
    mesh = plsc.VectorSubcoreMesh(core_axis_name="c", subcore_axis_name="s")

    @functools.partial(
        pl.kernel,
        out_type=jax.ShapeDtypeStruct((NC, ACC_ROWS, D), jnp.float32),
        mesh=mesh,
        scratch_types=[
            pltpu.VMEM((1, CHUNK), jnp.int32),
            pltpu.VMEM((1, CHUNK), jnp.int32),
            pltpu.VMEM((1, CHUNK, D), jnp.float32),
            pltpu.VMEM((ZROWS, D), jnp.float32),
            pltpu.VMEM_SHARED((ACC_ROWS, D), jnp.float32),
            pltpu.SemaphoreType.DMA,
        ],
    )
    def sc_aggregate(src_hbm, dst_hbm, h_hbm, out_hbm,
                     srcb, dstb, rows, zbuf, acc, sem):
        cid = lax.axis_index("c")
        sid = lax.axis_index("s")
        wid = sid * NC + cid

        zero16 = jnp.zeros((16,), jnp.float32)

        @pl.loop(0, ZROWS)
        def _(r):
            for j in range(D // 16):
                zbuf[r, pl.ds(j * 16, 16)] = zero16

        rows_per_tile = ACC_ROWS // NS

        @pl.loop(0, rows_per_tile // ZROWS)
        def _(z):
            pltpu.sync_copy(
                zbuf, acc.at[pl.ds(sid * rows_per_tile + z * ZROWS, ZROWS)])

        plsc.subcore_barrier()

        ebase = wid * EPW

        @pl.loop(0, CPW)
        def _(c):
            base = ebase + c * CHUNK
            pltpu.sync_copy(src_hbm.at[pl.ds(base, CHUNK)], srcb.at[0])
            pltpu.sync_copy(dst_hbm.at[pl.ds(base, CHUNK)], dstb.at[0])
            pltpu.async_copy(h_hbm.at[srcb.at[0]], rows.at[0], sem).wait()
            pltpu.sync_copy(rows.at[0], acc.at[dstb.at[0]], add=True)

        plsc.subcore_barrier()

        pltpu.sync_copy(
            acc.at[pl.ds(sid * rows_per_tile, rows_per_tile)],
            out_hbm.at[cid, pl.ds(sid * rows_per_tile, rows_per_tile)])

    return sc_aggregate


_sc_aggregate = _sc_aggregate_build()


def _layer_body(relu, p_ref, h_ref, wrel_ref, b_ref, wroot_ref, o_ref):
    aggr = p_ref[0] + p_ref[1]
    out = jnp.dot(aggr, wrel_ref[...], preferred_element_type=jnp.float32)
    out = out + jnp.dot(h_ref[...], wroot_ref[...],
                        preferred_element_type=jnp.float32)
    out = out + b_ref[...]
    if relu:
        out = jnp.maximum(out, 0.0)
    o_ref[...] = out


def _tc_layer(P, h, W_rel, b, W_root, relu):
    blk = 1000
    nblk = N // blk
    return pl.pallas_call(
        functools.partial(_layer_body, relu),
        grid=(nblk,),
        in_specs=[
            pl.BlockSpec((NC, blk, D), lambda i: (0, i, 0)),
            pl.BlockSpec((blk, D), lambda i: (i, 0)),
            pl.BlockSpec((D, D), lambda i: (0, 0)),
            pl.BlockSpec((1, D), lambda i: (0, 0)),
            pl.BlockSpec((D, D), lambda i: (0, 0)),
        ],
        out_specs=pl.BlockSpec((blk, D), lambda i: (i, 0)),
        out_shape=jax.ShapeDtypeStruct((N, D), jnp.float32),
    )(P, h, W_rel, b.reshape(1, D), W_root)


def _pool_body(nblk, batch_ref, h_ref, wl_ref, bl_ref, o_ref, acc_ref, cnt_ref):
    i = pl.program_id(0)

    @pl.when(i == 0)
    def _():
        acc_ref[...] = jnp.zeros_like(acc_ref)
        cnt_ref[...] = jnp.zeros_like(cnt_ref)

    ids = batch_ref[0, 0, :]
    blk = ids.shape[0]
    onehot = (ids[:, None] ==
              lax.broadcasted_iota(jnp.int32, (blk, G), 1)).astype(jnp.float32)
    acc_ref[...] += lax.dot_general(
        onehot, h_ref[...], (((0,), (0,)), ((), ())),
        preferred_element_type=jnp.float32)
    cnt_ref[...] += jnp.sum(onehot, axis=0)[:, None]

    @pl.when(i == nblk - 1)
    def _():
        pooled = acc_ref[...] / jnp.maximum(cnt_ref[...], 1.0)
        o_ref[...] = jnp.dot(pooled, wl_ref[...],
                             preferred_element_type=jnp.float32) + bl_ref[...]


def _tc_pool(h, batch, Wl, bl):
    blk = 1000
    nblk = N // blk
    C = Wl.shape[1]
    return pl.pallas_call(
        functools.partial(_pool_body, nblk),
        grid=(nblk,),
        in_specs=[
            pl.BlockSpec((1, 1, blk), lambda i: (i, 0, 0)),
            pl.BlockSpec((blk, D), lambda i: (i, 0)),
            pl.BlockSpec((D, C), lambda i: (0, 0)),
            pl.BlockSpec((1, C), lambda i: (0, 0)),
        ],
        out_specs=pl.BlockSpec((G, C), lambda i: (0, 0)),
        out_shape=jax.ShapeDtypeStruct((G, C), jnp.float32),
        scratch_shapes=[
            pltpu.VMEM((G, D), jnp.float32),
            pltpu.VMEM((G, D), jnp.float32),
        ],
    )(batch.reshape(N // blk, 1, blk), h, Wl, bl.reshape(1, C))


def kernel(x, edge_index, batch, W1_rel, b1, W1_root, W2_rel, b2, W2_root,
           W3_rel, b3, W3_root, Wl, bl):
    pad = E_PAD - E
    src = jnp.concatenate([edge_index[0], jnp.zeros((pad,), jnp.int32)])
    dst = jnp.concatenate(
        [edge_index[1], jnp.full((pad,), DEAD_ROW, jnp.int32)])

    h = x
    for W_rel, b, W_root, relu in (
            (W1_rel, b1, W1_root, True),
            (W2_rel, b2, W2_root, True),
            (W3_rel, b3, W3_root, False)):
        P = _sc_aggregate(src, dst, h)
        h = _tc_layer(P, h, W_rel, b, W_root, relu)

    return _tc_pool(h, batch, Wl, bl)
